# baseline (speedup 1.0000x reference)
import functools

import jax
import jax.numpy as jnp
from jax import lax
from jax.experimental import pallas as pl
from jax.experimental.pallas import tpu as pltpu
from jax.experimental.pallas import tpu_sc as plsc

N0, N1, N2 = 30000, 8000, 2000
D = 128
NID_PAD = 32768
E0P, E1P, E2P = 491520, 131072, 32768
ACC0, ACC1, ACC2 = 30720, 8192, 2048
NW = 32


def _mesh():
    return plsc.VectorSubcoreMesh(core_axis_name="c", subcore_axis_name="s")


def _sc_gather(y, nid2):
    blocks = NID_PAD // (NW * 128)

    @functools.partial(
        pl.kernel, mesh=_mesh(),
        compiler_params=pltpu.CompilerParams(use_tc_tiling_on_sc=False),
        out_type=jax.ShapeDtypeStruct((NID_PAD, 16), jnp.float32),
        scratch_types=[
            pltpu.VMEM((blocks, 128), jnp.int32),
            pltpu.VMEM((2, 128, 16), jnp.float32),
            pltpu.SemaphoreType.DMA,
            pltpu.SemaphoreType.DMA,
        ])
    def k(y_h, nid_h, t_h, idx_v, rows_v, gsem, ssem):
        wid = lax.axis_index("c") * 16 + lax.axis_index("s")
        pltpu.sync_copy(nid_h.at[pl.ds(wid * blocks, blocks)], idx_v)
        pltpu.async_copy(y_h.at[idx_v.at[0]], rows_v.at[0], gsem)

        def body(b, c):
            s = lax.rem(b, 2)
            pltpu.make_async_copy(y_h.at[idx_v.at[b]], rows_v.at[s], gsem).wait()

            @pl.when(b > 0)
            def _():
                pltpu.make_async_copy(
                    rows_v.at[1 - s],
                    t_h.at[pl.ds((wid * blocks + b - 1) * 128, 128)],
                    ssem).wait()

            @pl.when(b < blocks - 1)
            def _():
                pltpu.async_copy(y_h.at[idx_v.at[b + 1]], rows_v.at[1 - s], gsem)

            pltpu.async_copy(
                rows_v.at[s], t_h.at[pl.ds((wid * blocks + b) * 128, 128)], ssem)
            return c

        lax.fori_loop(0, blocks, body, 0)
        pltpu.make_async_copy(
            rows_v.at[(blocks - 1) % 2],
            t_h.at[pl.ds((wid * blocks + blocks - 1) * 128, 128)], ssem).wait()

    return k(y, nid2)


def _sc_segsum(msg, src2, dst2, zeros, n_acc, width, ep, nbuf):
    blocks = ep // (NW * 128)
    groups = blocks // nbuf
    rpt = n_acc // 16

    @functools.partial(
        pl.kernel, mesh=_mesh(),
        compiler_params=pltpu.CompilerParams(use_tc_tiling_on_sc=False),
        out_type=jax.ShapeDtypeStruct((2, n_acc, width), jnp.float32),
        scratch_types=[
            pltpu.VMEM((blocks, 128), jnp.int32),
            pltpu.VMEM((blocks, 128), jnp.int32),
            pltpu.VMEM((2, nbuf, 128, width), jnp.float32),
            pltpu.VMEM_SHARED((n_acc, width), jnp.float32),
            pltpu.SemaphoreType.DMA,
            pltpu.SemaphoreType.DMA,
        ])
    def k(msg_h, src_h, dst_h, zero_h, out_h, src_i, dst_i, rows_v, acc_sh,
          gsem, ssem):
        cid = lax.axis_index("c")
        sid = lax.axis_index("s")
        wid = cid * 16 + sid
        r0 = sid * rpt
        pltpu.sync_copy(src_h.at[pl.ds(wid * blocks, blocks)], src_i)
        pltpu.sync_copy(dst_h.at[pl.ds(wid * blocks, blocks)], dst_i)
        pltpu.sync_copy(zero_h.at[pl.ds(r0, rpt)], acc_sh.at[pl.ds(r0, rpt)])
        plsc.subcore_barrier()

        for j in range(nbuf):
            pltpu.async_copy(msg_h.at[src_i.at[j]], rows_v.at[0, j], gsem)

        def giter(g, c):
            s = lax.rem(g, 2)
            base = g * nbuf
            for j in range(nbuf):
                pltpu.make_async_copy(
                    msg_h.at[src_i.at[base + j]], rows_v.at[s, j], gsem).wait()

            @pl.when(g > 0)
            def _():
                for j in range(nbuf):
                    pltpu.make_async_copy(
                        rows_v.at[1 - s, j],
                        acc_sh.at[dst_i.at[base - nbuf + j]], ssem).wait()

            @pl.when(g < groups - 1)
            def _():
                for j in range(nbuf):
                    pltpu.async_copy(
                        msg_h.at[src_i.at[base + nbuf + j]],
                        rows_v.at[1 - s, j], gsem)

            for j in range(nbuf):
                pltpu.async_copy(
                    rows_v.at[s, j], acc_sh.at[dst_i.at[base + j]], ssem,
                    add=True)
            return c

        lax.fori_loop(0, groups, giter, 0)
        sl = (groups - 1) % 2
        for j in range(nbuf):
            pltpu.make_async_copy(
                rows_v.at[sl, j],
                acc_sh.at[dst_i.at[(groups - 1) * nbuf + j]], ssem).wait()

        plsc.subcore_barrier()
        pltpu.sync_copy(acc_sh.at[pl.ds(r0, rpt)],
                        out_h.at[cid, pl.ds(r0, rpt)])

    return k(msg, src2, dst2, zeros)


def _tc_proj(x2, M1):
    R = 10000

    def k(x_ref, m_ref, o_ref):
        y = jnp.dot(x_ref[...], m_ref[...], preferred_element_type=jnp.float32)
        col = lax.broadcasted_iota(jnp.int32, y.shape, 1)
        o_ref[...] = jnp.where(col == 6, 1.0, y)

    n = x2.shape[0]
    return pl.pallas_call(
        k, grid=(n // R,),
        in_specs=[pl.BlockSpec((R, D), lambda i: (i, 0)),
                  pl.BlockSpec((D, 16), lambda i: (0, 0))],
        out_specs=pl.BlockSpec((R, 16), lambda i: (i, 0)),
        out_shape=jax.ShapeDtypeStruct((n, 16), jnp.float32))(x2, M1)


def _tc_d(acc1p, T1p, C6, Sh, S16, MK, B1p):
    RP = ACC0 // 8 // 3

    def k(a_ref, t_ref, c_ref, sh_ref, s_ref, mk_ref, b_ref, o_ref):
        a = a_ref[0] + a_ref[1]
        cnt = jnp.maximum(
            jnp.dot(a, c_ref[...], preferred_element_type=jnp.float32), 1.0)
        pre = (a / cnt * mk_ref[0:1, :]
               + jnp.dot(t_ref[...], sh_ref[...],
                         preferred_element_type=jnp.float32)
               + b_ref[0:1, :])
        ss = jnp.dot(pre * pre, s_ref[...], preferred_element_type=jnp.float32)
        n = jnp.maximum(jnp.sqrt(ss), 1e-12)
        o_ref[...] = jnp.maximum(pre / n, 0.0)

    return pl.pallas_call(
        k, grid=(ACC0 // 8 // RP,),
        in_specs=[pl.BlockSpec((2, RP, 128), lambda i: (0, i, 0)),
                  pl.BlockSpec((RP, 128), lambda i: (i, 0)),
                  pl.BlockSpec((128, 128), lambda i: (0, 0)),
                  pl.BlockSpec((128, 128), lambda i: (0, 0)),
                  pl.BlockSpec((128, 128), lambda i: (0, 0)),
                  pl.BlockSpec((8, 128), lambda i: (0, 0)),
                  pl.BlockSpec((8, 128), lambda i: (0, 0))],
        out_specs=pl.BlockSpec((RP, 128), lambda i: (i, 0)),
        out_shape=jax.ShapeDtypeStruct((ACC0 // 8, 128), jnp.float32))(
            acc1p, T1p, C6, Sh, S16, MK, B1p)


def _tc_f(acc2p, acc1p, T2p, C6, W2Ap, W2Bp, S48, B2p):
    RP = ACC0 // 8 // 3

    def k(a2_ref, a1_ref, t_ref, c_ref, wa_ref, wb_ref, s_ref, b_ref, o_ref):
        a1 = a1_ref[0] + a1_ref[1]
        inv = 1.0 / jnp.maximum(
            jnp.dot(a1, c_ref[...], preferred_element_type=jnp.float32), 1.0)
        a2 = (a2_ref[0] + a2_ref[1]) * inv
        h = (jnp.dot(t_ref[...], wa_ref[...],
                     preferred_element_type=jnp.float32)
             + jnp.dot(a2, wb_ref[...], preferred_element_type=jnp.float32)
             + b_ref[0:1, :])
        ss = jnp.dot(h * h, s_ref[...], preferred_element_type=jnp.float32)
        n = jnp.maximum(jnp.sqrt(ss), 1e-12)
        h = jnp.maximum(h / n, 0.0)
        col = lax.broadcasted_iota(jnp.int32, h.shape, 1)
        o_ref[...] = jnp.where(col % 48 == 36, 1.0, h)

    return pl.pallas_call(
        k, grid=(ACC0 // 8 // RP,),
        in_specs=[pl.BlockSpec((2, RP, 128), lambda i: (0, i, 0)),
                  pl.BlockSpec((2, RP, 128), lambda i: (0, i, 0)),
                  pl.BlockSpec((RP, 128), lambda i: (i, 0)),
                  pl.BlockSpec((128, 128), lambda i: (0, 0)),
                  pl.BlockSpec((128, 384), lambda i: (0, 0)),
                  pl.BlockSpec((128, 384), lambda i: (0, 0)),
                  pl.BlockSpec((384, 384), lambda i: (0, 0)),
                  pl.BlockSpec((8, 384), lambda i: (0, 0))],
        out_specs=pl.BlockSpec((RP, 384), lambda i: (i, 0)),
        out_shape=jax.ShapeDtypeStruct((ACC0 // 8, 384), jnp.float32))(
            acc2p, acc1p, T2p, C6, W2Ap, W2Bp, S48, B2p)


def _tc_h(acc3, M3, B3, M4):
    R = 2048

    def k(a_ref, m3_ref, b3_ref, m4_ref, o_ref):
        a = a_ref[0] + a_ref[1]
        a = a / jnp.clip(a[:, 36:37], 1.0)
        h3 = jnp.maximum(
            jnp.dot(a, m3_ref[...], preferred_element_type=jnp.float32)
            + b3_ref[0:1, :], 0.0)
        g = jnp.dot(h3, m4_ref[...], preferred_element_type=jnp.float32)
        col = lax.broadcasted_iota(jnp.int32, g.shape, 1)
        o_ref[...] = jnp.where(col == 128, 1.0, g)

    return pl.pallas_call(
        k, grid=(ACC1 // R,),
        in_specs=[pl.BlockSpec((2, R, 48), lambda i: (0, i, 0)),
                  pl.BlockSpec((48, 224), lambda i: (0, 0)),
                  pl.BlockSpec((8, 224), lambda i: (0, 0)),
                  pl.BlockSpec((224, 144), lambda i: (0, 0))],
        out_specs=pl.BlockSpec((R, 144), lambda i: (i, 0)),
        out_shape=jax.ShapeDtypeStruct((ACC1, 144), jnp.float32))(
            acc3, M3, B3, M4)


def _tc_j(acc4, B4):
    def k(a_ref, b_ref, o_ref):
        a = a_ref[0] + a_ref[1]
        cnt = jnp.clip(a[:, 128:129], 1.0)
        o = a[:, 0:128] / cnt + b_ref[0:1, :]
        o_ref[...] = o[0:N2][None]

    return pl.pallas_call(
        k, grid=(1,),
        in_specs=[pl.BlockSpec((2, ACC2, 144), lambda i: (0, 0, 0)),
                  pl.BlockSpec((8, 128), lambda i: (0, 0))],
        out_specs=pl.BlockSpec((1, N2, 128), lambda i: (0, 0, 0)),
        out_shape=jax.ShapeDtypeStruct((1, N2, 128), jnp.float32))(acc4, B4)


def kernel(x, n_id, edge_index0, edge_index1, edge_index2, res_n_id1,
           res_n_id2, W1, b1, W2, b2, W3, b3, W4, b4):
    i32 = jnp.int32
    f32 = jnp.float32
    x2 = x.reshape(x.shape[1], x.shape[2])

    nid_p = jnp.concatenate(
        [n_id.astype(i32), jnp.zeros((NID_PAD - N0,), i32)]).reshape(-1, 128)

    def pad_e(ei, ep, dummy):
        e = ei.shape[1]
        s = jnp.concatenate([ei[0].astype(i32), jnp.zeros((ep - e,), i32)])
        d = jnp.concatenate([ei[1].astype(i32), jnp.full((ep - e,), dummy, i32)])
        return s.reshape(-1, 128), d.reshape(-1, 128)

    e0s, e0d = pad_e(edge_index0, E0P, N0)
    e1s, e1d = pad_e(edge_index1, E1P, N1)
    e2s, e2d = pad_e(edge_index2, E2P, N2)
    z0 = jnp.zeros((ACC0, 16), f32)
    z1 = jnp.zeros((ACC1, 48), f32)
    z2 = jnp.zeros((ACC2, 144), f32)

    M1 = jnp.zeros((D, 16), f32).at[:, 0:6].set(W1[D:]).at[:, 8:14].set(W1[:D])
    M3 = jnp.zeros((48, 224), f32).at[0:36, 0:216].set(W3)
    B3 = jnp.zeros((8, 224), f32).at[0, 0:216].set(b3)
    M4 = jnp.zeros((224, 144), f32).at[0:216, 0:128].set(W4)
    B4 = jnp.zeros((8, 128), f32).at[0].set(b4)

    li = jnp.arange(128)
    g16, l16 = li // 16, li % 16
    same16 = g16[:, None] == g16[None, :]
    C6 = jnp.where((l16[:, None] == 6) & same16, 1.0, 0.0)
    Sh = jnp.where(same16 & (l16[:, None] == l16[None, :] + 8)
                   & (l16[None, :] < 6), 1.0, 0.0)
    S16 = jnp.where(same16, 1.0, 0.0)
    MK = jnp.zeros((8, 128), f32).at[0].set(jnp.where(l16 < 6, 1.0, 0.0))
    B1p = jnp.zeros((8, 128), f32).at[0].set(
        jnp.where(l16 < 6, b1[jnp.minimum(l16, 5)], 0.0))
    lj = jnp.arange(384)
    g48, l48 = lj // 48, lj % 48
    match = (g16[:, None] == g48[None, :]) & (l16[:, None] < 6) \
        & (l48[None, :] < 36)
    W2sub = W2[jnp.minimum(l16, 5)[:, None], jnp.minimum(l48, 35)[None, :]]
    W2Ap = jnp.where(match, W2sub, 0.0)
    W2Bp = jnp.where(
        match,
        W2[6 + jnp.minimum(l16, 5)[:, None], jnp.minimum(l48, 35)[None, :]],
        0.0)
    S48 = jnp.where(g48[:, None] == g48[None, :], 1.0, 0.0)
    B2p = jnp.zeros((8, 384), f32).at[0].set(
        jnp.where(l48 < 36, b2[jnp.minimum(l48, 35)], 0.0))

    Y = _tc_proj(x2, M1)
    T1 = _sc_gather(Y, nid_p)
    T1p = T1.reshape(-1, 128)
    acc1 = _sc_segsum(T1, e0s, e0d, z0, ACC0, 16, E0P, 6)
    acc1p = acc1.reshape(2, -1, 128)
    T2p = _tc_d(acc1p, T1p, C6, Sh, S16, MK, B1p)
    acc2 = _sc_segsum(T2p.reshape(-1, 16), e0s, e0d, z0, ACC0, 16, E0P, 6)
    T3p = _tc_f(acc2.reshape(2, -1, 128), acc1p, T2p, C6, W2Ap, W2Bp, S48, B2p)
    acc3 = _sc_segsum(T3p.reshape(-1, 48), e1s, e1d, z1, ACC1, 48, E1P, 4)
    T4 = _tc_h(acc3, M3, B3, M4)
    acc4 = _sc_segsum(T4, e2s, e2d, z2, ACC2, 144, E2P, 2)
    return _tc_j(acc4, B4)


# speedup vs baseline: 2.4436x; 2.4436x over previous
import functools

import jax
import jax.numpy as jnp
from jax import lax
from jax.experimental import pallas as pl
from jax.experimental.pallas import tpu as pltpu
from jax.experimental.pallas import tpu_sc as plsc

N0, N1, N2 = 30000, 8000, 2000
D = 128
NID_PAD = 32768
E0P, E1P, E2P = 491520, 131072, 32768
ACC0, ACC1, ACC2 = 30720, 8192, 2048
NW = 32


def _mesh():
    return plsc.VectorSubcoreMesh(core_axis_name="c", subcore_axis_name="s")


def _sc_gather(y, nid2):
    blocks = NID_PAD // (NW * 128)

    @functools.partial(
        pl.kernel, mesh=_mesh(),
        compiler_params=pltpu.CompilerParams(use_tc_tiling_on_sc=False),
        out_type=jax.ShapeDtypeStruct((NID_PAD, 16), jnp.float32),
        scratch_types=[
            pltpu.VMEM((blocks, 128), jnp.int32),
            pltpu.VMEM((2, 128, 16), jnp.float32),
            pltpu.SemaphoreType.DMA,
            pltpu.SemaphoreType.DMA,
        ])
    def k(y_h, nid_h, t_h, idx_v, rows_v, gsem, ssem):
        wid = lax.axis_index("c") * 16 + lax.axis_index("s")
        pltpu.sync_copy(nid_h.at[pl.ds(wid * blocks, blocks)], idx_v)
        pltpu.async_copy(y_h.at[idx_v.at[0]], rows_v.at[0], gsem)

        def body(b, c):
            s = lax.rem(b, 2)
            pltpu.make_async_copy(y_h.at[idx_v.at[b]], rows_v.at[s], gsem).wait()

            @pl.when(b > 0)
            def _():
                pltpu.make_async_copy(
                    rows_v.at[1 - s],
                    t_h.at[pl.ds((wid * blocks + b - 1) * 128, 128)],
                    ssem).wait()

            @pl.when(b < blocks - 1)
            def _():
                pltpu.async_copy(y_h.at[idx_v.at[b + 1]], rows_v.at[1 - s], gsem)

            pltpu.async_copy(
                rows_v.at[s], t_h.at[pl.ds((wid * blocks + b) * 128, 128)], ssem)
            return c

        lax.fori_loop(0, blocks, body, 0)
        pltpu.make_async_copy(
            rows_v.at[(blocks - 1) % 2],
            t_h.at[pl.ds((wid * blocks + blocks - 1) * 128, 128)], ssem).wait()

    return k(y, nid2)


def _sc_segsum(msg, src2, dst2, zeros, n_acc, width, ep, nbuf):
    blocks = ep // (NW * 128)
    groups = blocks // nbuf
    rpt = n_acc // 16

    @functools.partial(
        pl.kernel, mesh=_mesh(),
        compiler_params=pltpu.CompilerParams(use_tc_tiling_on_sc=False),
        out_type=jax.ShapeDtypeStruct((2, n_acc, width), jnp.float32),
        scratch_types=[
            pltpu.VMEM((blocks, 128), jnp.int32),
            pltpu.VMEM((blocks, 128), jnp.int32),
            pltpu.VMEM((2, nbuf, 128, width), jnp.float32),
            pltpu.VMEM_SHARED((n_acc, width), jnp.float32),
            pltpu.SemaphoreType.DMA,
            pltpu.SemaphoreType.DMA,
        ])
    def k(msg_h, src_h, dst_h, zero_h, out_h, src_i, dst_i, rows_v, acc_sh,
          gsem, ssem):
        cid = lax.axis_index("c")
        sid = lax.axis_index("s")
        wid = cid * 16 + sid
        r0 = sid * rpt
        pltpu.sync_copy(src_h.at[pl.ds(wid * blocks, blocks)], src_i)
        pltpu.sync_copy(dst_h.at[pl.ds(wid * blocks, blocks)], dst_i)
        pltpu.sync_copy(zero_h.at[pl.ds(r0, rpt)], acc_sh.at[pl.ds(r0, rpt)])
        plsc.subcore_barrier()

        for j in range(nbuf):
            pltpu.async_copy(msg_h.at[src_i.at[j]], rows_v.at[0, j], gsem)

        def giter(g, c):
            s = lax.rem(g, 2)
            base = g * nbuf
            for j in range(nbuf):
                pltpu.make_async_copy(
                    msg_h.at[src_i.at[base + j]], rows_v.at[s, j], gsem).wait()

            @pl.when(g > 0)
            def _():
                for j in range(nbuf):
                    pltpu.make_async_copy(
                        rows_v.at[1 - s, j],
                        acc_sh.at[dst_i.at[base - nbuf + j]], ssem).wait()

            @pl.when(g < groups - 1)
            def _():
                for j in range(nbuf):
                    pltpu.async_copy(
                        msg_h.at[src_i.at[base + nbuf + j]],
                        rows_v.at[1 - s, j], gsem)

            for j in range(nbuf):
                pltpu.async_copy(
                    rows_v.at[s, j], acc_sh.at[dst_i.at[base + j]], ssem,
                    add=True)
            return c

        lax.fori_loop(0, groups, giter, 0)
        sl = (groups - 1) % 2
        for j in range(nbuf):
            pltpu.make_async_copy(
                rows_v.at[sl, j],
                acc_sh.at[dst_i.at[(groups - 1) * nbuf + j]], ssem).wait()

        plsc.subcore_barrier()
        pltpu.sync_copy(acc_sh.at[pl.ds(r0, rpt)],
                        out_h.at[cid, pl.ds(r0, rpt)])

    return k(msg, src2, dst2, zeros)


def _tc_proj(x2, M1):
    R = 10000

    def k(x_ref, m_ref, o_ref):
        y = jnp.dot(x_ref[...], m_ref[...], preferred_element_type=jnp.float32)
        col = lax.broadcasted_iota(jnp.int32, y.shape, 1)
        o_ref[...] = jnp.where(col == 6, 1.0, y)

    n = x2.shape[0]
    return pl.pallas_call(
        k, grid=(n // R,),
        in_specs=[pl.BlockSpec((R, D), lambda i: (i, 0)),
                  pl.BlockSpec((D, 16), lambda i: (0, 0))],
        out_specs=pl.BlockSpec((R, 16), lambda i: (i, 0)),
        out_shape=jax.ShapeDtypeStruct((n, 16), jnp.float32))(x2, M1)


def _tc_d(acc1p, T1p, C6, Sh, S16, MK, B1p):
    RP = ACC0 // 8 // 3

    def k(a_ref, t_ref, c_ref, sh_ref, s_ref, mk_ref, b_ref, o_ref):
        a = a_ref[0] + a_ref[1]
        cnt = jnp.maximum(
            jnp.dot(a, c_ref[...], preferred_element_type=jnp.float32), 1.0)
        pre = (a / cnt * mk_ref[0:1, :]
               + jnp.dot(t_ref[...], sh_ref[...],
                         preferred_element_type=jnp.float32)
               + b_ref[0:1, :])
        ss = jnp.dot(pre * pre, s_ref[...], preferred_element_type=jnp.float32)
        n = jnp.maximum(jnp.sqrt(ss), 1e-12)
        o_ref[...] = jnp.maximum(pre / n, 0.0)

    return pl.pallas_call(
        k, grid=(ACC0 // 8 // RP,),
        in_specs=[pl.BlockSpec((2, RP, 128), lambda i: (0, i, 0)),
                  pl.BlockSpec((RP, 128), lambda i: (i, 0)),
                  pl.BlockSpec((128, 128), lambda i: (0, 0)),
                  pl.BlockSpec((128, 128), lambda i: (0, 0)),
                  pl.BlockSpec((128, 128), lambda i: (0, 0)),
                  pl.BlockSpec((8, 128), lambda i: (0, 0)),
                  pl.BlockSpec((8, 128), lambda i: (0, 0))],
        out_specs=pl.BlockSpec((RP, 128), lambda i: (i, 0)),
        out_shape=jax.ShapeDtypeStruct((ACC0 // 8, 128), jnp.float32))(
            acc1p, T1p, C6, Sh, S16, MK, B1p)


def _tc_f(acc2p, acc1p, T2p, C6, W2Ap, W2Bp, S48, B2p):
    RP = ACC0 // 8 // 3

    def k(a2_ref, a1_ref, t_ref, c_ref, wa_ref, wb_ref, s_ref, b_ref, o_ref):
        a1 = a1_ref[0] + a1_ref[1]
        inv = 1.0 / jnp.maximum(
            jnp.dot(a1, c_ref[...], preferred_element_type=jnp.float32), 1.0)
        a2 = (a2_ref[0] + a2_ref[1]) * inv
        h = (jnp.dot(t_ref[...], wa_ref[...],
                     preferred_element_type=jnp.float32)
             + jnp.dot(a2, wb_ref[...], preferred_element_type=jnp.float32)
             + b_ref[0:1, :])
        ss = jnp.dot(h * h, s_ref[...], preferred_element_type=jnp.float32)
        n = jnp.maximum(jnp.sqrt(ss), 1e-12)
        h = jnp.maximum(h / n, 0.0)
        col = lax.broadcasted_iota(jnp.int32, h.shape, 1)
        o_ref[...] = jnp.where(col % 48 == 36, 1.0, h)

    return pl.pallas_call(
        k, grid=(ACC0 // 8 // RP,),
        in_specs=[pl.BlockSpec((2, RP, 128), lambda i: (0, i, 0)),
                  pl.BlockSpec((2, RP, 128), lambda i: (0, i, 0)),
                  pl.BlockSpec((RP, 128), lambda i: (i, 0)),
                  pl.BlockSpec((128, 128), lambda i: (0, 0)),
                  pl.BlockSpec((128, 384), lambda i: (0, 0)),
                  pl.BlockSpec((128, 384), lambda i: (0, 0)),
                  pl.BlockSpec((384, 384), lambda i: (0, 0)),
                  pl.BlockSpec((8, 384), lambda i: (0, 0))],
        out_specs=pl.BlockSpec((RP, 384), lambda i: (i, 0)),
        out_shape=jax.ShapeDtypeStruct((ACC0 // 8, 384), jnp.float32))(
            acc2p, acc1p, T2p, C6, W2Ap, W2Bp, S48, B2p)


def _tc_h(acc3, M3, B3, M4):
    R = 2048

    def k(a_ref, m3_ref, b3_ref, m4_ref, o_ref):
        a = a_ref[0] + a_ref[1]
        a = a / jnp.clip(a[:, 36:37], 1.0)
        h3 = jnp.maximum(
            jnp.dot(a, m3_ref[...], preferred_element_type=jnp.float32)
            + b3_ref[0:1, :], 0.0)
        g = jnp.dot(h3, m4_ref[...], preferred_element_type=jnp.float32)
        col = lax.broadcasted_iota(jnp.int32, g.shape, 1)
        o_ref[...] = jnp.where(col == 128, 1.0, g)

    return pl.pallas_call(
        k, grid=(ACC1 // R,),
        in_specs=[pl.BlockSpec((2, R, 48), lambda i: (0, i, 0)),
                  pl.BlockSpec((48, 224), lambda i: (0, 0)),
                  pl.BlockSpec((8, 224), lambda i: (0, 0)),
                  pl.BlockSpec((224, 144), lambda i: (0, 0))],
        out_specs=pl.BlockSpec((R, 144), lambda i: (i, 0)),
        out_shape=jax.ShapeDtypeStruct((ACC1, 144), jnp.float32))(
            acc3, M3, B3, M4)


def _tc_j(acc4, B4):
    def k(a_ref, b_ref, o_ref):
        a = a_ref[0] + a_ref[1]
        cnt = jnp.clip(a[:, 128:129], 1.0)
        o = a[:, 0:128] / cnt + b_ref[0:1, :]
        o_ref[...] = o[0:N2][None]

    return pl.pallas_call(
        k, grid=(1,),
        in_specs=[pl.BlockSpec((2, ACC2, 144), lambda i: (0, 0, 0)),
                  pl.BlockSpec((8, 128), lambda i: (0, 0))],
        out_specs=pl.BlockSpec((1, N2, 128), lambda i: (0, 0, 0)),
        out_shape=jax.ShapeDtypeStruct((1, N2, 128), jnp.float32))(acc4, B4)


def kernel(x, n_id, edge_index0, edge_index1, edge_index2, res_n_id1,
           res_n_id2, W1, b1, W2, b2, W3, b3, W4, b4):
    i32 = jnp.int32
    f32 = jnp.float32
    x2 = x.reshape(x.shape[1], x.shape[2])

    nid_p = jnp.concatenate(
        [n_id.astype(i32), jnp.zeros((NID_PAD - N0,), i32)]).reshape(-1, 128)

    def pad_e(ei, ep, dummy):
        e = ei.shape[1]
        s = jnp.concatenate([ei[0].astype(i32), jnp.zeros((ep - e,), i32)])
        d = jnp.concatenate([ei[1].astype(i32), jnp.full((ep - e,), dummy, i32)])
        return s.reshape(-1, 128), d.reshape(-1, 128)

    e0s, e0d = pad_e(edge_index0, E0P, N0)
    e1s, e1d = pad_e(edge_index1, E1P, N1)
    e2s, e2d = pad_e(edge_index2, E2P, N2)
    z0 = jnp.zeros((ACC0, 16), f32)
    z1 = jnp.zeros((ACC1, 48), f32)
    z2 = jnp.zeros((ACC2, 144), f32)

    M1 = jnp.zeros((D, 16), f32).at[:, 0:6].set(W1[D:]).at[:, 8:14].set(W1[:D])
    M3 = jnp.zeros((48, 224), f32).at[0:36, 0:216].set(W3)
    B3 = jnp.zeros((8, 224), f32).at[0, 0:216].set(b3)
    M4 = jnp.zeros((224, 144), f32).at[0:216, 0:128].set(W4)
    B4 = jnp.zeros((8, 128), f32).at[0].set(b4)

    li = jnp.arange(128)
    g16, l16 = li // 16, li % 16
    same16 = g16[:, None] == g16[None, :]
    C6 = jnp.where((l16[:, None] == 6) & same16, 1.0, 0.0)
    Sh = jnp.where(same16 & (l16[:, None] == l16[None, :] + 8)
                   & (l16[None, :] < 6), 1.0, 0.0)
    S16 = jnp.where(same16, 1.0, 0.0)
    MK = jnp.zeros((8, 128), f32).at[0].set(jnp.where(l16 < 6, 1.0, 0.0))
    lj = jnp.arange(384)
    g48, l48 = lj // 48, lj % 48
    P16 = jnp.where(l16[:, None] == jnp.arange(6)[None, :], 1.0, 0.0)
    P48 = jnp.where(l48[:, None] == jnp.arange(36)[None, :], 1.0, 0.0)
    gmask = jnp.where(g16[:, None] == g48[None, :], 1.0, 0.0)
    W2Ap = (P16 @ W2[:6] @ P48.T) * gmask
    W2Bp = (P16 @ W2[6:] @ P48.T) * gmask
    S48 = jnp.where(g48[:, None] == g48[None, :], 1.0, 0.0)
    B1p = jnp.zeros((8, 128), f32).at[0].set(P16 @ b1)
    B2p = jnp.zeros((8, 384), f32).at[0].set(P48 @ b2)

    Y = _tc_proj(x2, M1)
    T1 = _sc_gather(Y, nid_p)
    T1p = T1.reshape(-1, 128)
    acc1 = _sc_segsum(T1, e0s, e0d, z0, ACC0, 16, E0P, 6)
    acc1p = acc1.reshape(2, -1, 128)
    T2p = _tc_d(acc1p, T1p, C6, Sh, S16, MK, B1p)
    acc2 = _sc_segsum(T2p.reshape(-1, 16), e0s, e0d, z0, ACC0, 16, E0P, 6)
    T3p = _tc_f(acc2.reshape(2, -1, 128), acc1p, T2p, C6, W2Ap, W2Bp, S48, B2p)
    acc3 = _sc_segsum(T3p.reshape(-1, 48), e1s, e1d, z1, ACC1, 48, E1P, 4)
    T4 = _tc_h(acc3, M3, B3, M4)
    acc4 = _sc_segsum(T4, e2s, e2d, z2, ACC2, 144, E2P, 2)
    return _tc_j(acc4, B4)


# speedup vs baseline: 2.5705x; 1.0519x over previous
import functools

import jax
import jax.numpy as jnp
from jax import lax
from jax.experimental import pallas as pl
from jax.experimental.pallas import tpu as pltpu
from jax.experimental.pallas import tpu_sc as plsc

N0, N1, N2 = 30000, 8000, 2000
D = 128
NID_PAD = 32768
E0P, E1P, E2P = 491520, 131072, 32768
ACC0, ACC1, ACC2 = 30720, 8192, 2048
NW = 32


def _mesh():
    return plsc.VectorSubcoreMesh(core_axis_name="c", subcore_axis_name="s")


def _sc_gather(y, nid2, b0, b1):
    bmax = max(b0, b1)

    @functools.partial(
        pl.kernel, mesh=_mesh(),
        compiler_params=pltpu.CompilerParams(use_tc_tiling_on_sc=False),
        out_type=jax.ShapeDtypeStruct((NID_PAD, 16), jnp.float32),
        scratch_types=[
            pltpu.VMEM((bmax, 128), jnp.int32),
            pltpu.VMEM((2, 128, 16), jnp.float32),
            pltpu.SemaphoreType.DMA,
            pltpu.SemaphoreType.DMA,
        ])
    def k(y_h, nid_h, t_h, idx_v, rows_v, gsem, ssem):
        cid = lax.axis_index("c")
        sid = lax.axis_index("s")
        off = lax.select(cid == 0, sid * b0, 16 * b0 + sid * b1)
        blocks = lax.select(cid == 0, b0, b1)
        pltpu.sync_copy(nid_h.at[pl.ds(off, bmax)], idx_v)
        pltpu.async_copy(y_h.at[idx_v.at[0]], rows_v.at[0], gsem)

        def body(b, c):
            s = lax.rem(b, 2)
            pltpu.make_async_copy(y_h.at[idx_v.at[b]], rows_v.at[s], gsem).wait()

            @pl.when(b > 0)
            def _():
                pltpu.make_async_copy(
                    rows_v.at[1 - s],
                    t_h.at[pl.ds((off + b - 1) * 128, 128)],
                    ssem).wait()

            @pl.when(b < blocks - 1)
            def _():
                pltpu.async_copy(y_h.at[idx_v.at[b + 1]], rows_v.at[1 - s], gsem)

            pltpu.async_copy(
                rows_v.at[s], t_h.at[pl.ds((off + b) * 128, 128)], ssem)
            return c

        lax.fori_loop(0, blocks, body, 0)
        pltpu.make_async_copy(
            rows_v.at[lax.rem(blocks - 1, 2)],
            t_h.at[pl.ds((off + blocks - 1) * 128, 128)], ssem).wait()

    return k(y, nid2)


def _sc_segsum(msg, src2, dst2, zeros, n_acc, width, b0, b1, nbuf):
    bmax = max(b0, b1)
    rpt = n_acc // 16

    @functools.partial(
        pl.kernel, mesh=_mesh(),
        compiler_params=pltpu.CompilerParams(use_tc_tiling_on_sc=False),
        out_type=jax.ShapeDtypeStruct((2, n_acc, width), jnp.float32),
        scratch_types=[
            pltpu.VMEM((bmax, 128), jnp.int32),
            pltpu.VMEM((bmax, 128), jnp.int32),
            pltpu.VMEM((2, nbuf, 128, width), jnp.float32),
            pltpu.VMEM_SHARED((n_acc, width), jnp.float32),
            pltpu.SemaphoreType.DMA,
            pltpu.SemaphoreType.DMA,
        ])
    def k(msg_h, src_h, dst_h, zero_h, out_h, src_i, dst_i, rows_v, acc_sh,
          gsem, ssem):
        cid = lax.axis_index("c")
        sid = lax.axis_index("s")
        off = lax.select(cid == 0, sid * b0, 16 * b0 + sid * b1)
        groups = lax.select(cid == 0, b0 // nbuf, b1 // nbuf)
        r0 = sid * rpt
        pltpu.sync_copy(src_h.at[pl.ds(off, bmax)], src_i)
        pltpu.sync_copy(dst_h.at[pl.ds(off, bmax)], dst_i)
        pltpu.sync_copy(zero_h.at[pl.ds(r0, rpt)], acc_sh.at[pl.ds(r0, rpt)])
        plsc.subcore_barrier()

        for j in range(nbuf):
            pltpu.async_copy(msg_h.at[src_i.at[j]], rows_v.at[0, j], gsem)

        def giter(g, c):
            s = lax.rem(g, 2)
            base = g * nbuf
            for j in range(nbuf):
                pltpu.make_async_copy(
                    msg_h.at[src_i.at[base + j]], rows_v.at[s, j], gsem).wait()

            @pl.when(g > 0)
            def _():
                for j in range(nbuf):
                    pltpu.make_async_copy(
                        rows_v.at[1 - s, j],
                        acc_sh.at[dst_i.at[base - nbuf + j]], ssem).wait()

            @pl.when(g < groups - 1)
            def _():
                for j in range(nbuf):
                    pltpu.async_copy(
                        msg_h.at[src_i.at[base + nbuf + j]],
                        rows_v.at[1 - s, j], gsem)

            for j in range(nbuf):
                pltpu.async_copy(
                    rows_v.at[s, j], acc_sh.at[dst_i.at[base + j]], ssem,
                    add=True)
            return c

        lax.fori_loop(0, groups, giter, 0)
        sl = lax.rem(groups - 1, 2)
        for j in range(nbuf):
            pltpu.make_async_copy(
                rows_v.at[sl, j],
                acc_sh.at[dst_i.at[(groups - 1) * nbuf + j]], ssem).wait()

        plsc.subcore_barrier()
        pltpu.sync_copy(acc_sh.at[pl.ds(r0, rpt)],
                        out_h.at[cid, pl.ds(r0, rpt)])

    return k(msg, src2, dst2, zeros)


def _tc_proj(x2, M1):
    R = 10000

    def k(x_ref, m_ref, o_ref):
        y = jnp.dot(x_ref[...], m_ref[...], preferred_element_type=jnp.float32)
        col = lax.broadcasted_iota(jnp.int32, y.shape, 1)
        o_ref[...] = jnp.where(col == 6, 1.0, y)

    n = x2.shape[0]
    return pl.pallas_call(
        k, grid=(n // R,),
        in_specs=[pl.BlockSpec((R, D), lambda i: (i, 0)),
                  pl.BlockSpec((D, 16), lambda i: (0, 0))],
        out_specs=pl.BlockSpec((R, 16), lambda i: (i, 0)),
        out_shape=jax.ShapeDtypeStruct((n, 16), jnp.float32))(x2, M1)


def _tc_d(acc1p, T1p, C6, Sh, S16, MK, B1p):
    RP = ACC0 // 8 // 3

    def k(a_ref, t_ref, c_ref, sh_ref, s_ref, mk_ref, b_ref, o_ref):
        a = a_ref[0] + a_ref[1]
        cnt = jnp.maximum(
            jnp.dot(a, c_ref[...], preferred_element_type=jnp.float32), 1.0)
        pre = (a / cnt * mk_ref[0:1, :]
               + jnp.dot(t_ref[...], sh_ref[...],
                         preferred_element_type=jnp.float32)
               + b_ref[0:1, :])
        ss = jnp.dot(pre * pre, s_ref[...], preferred_element_type=jnp.float32)
        n = jnp.maximum(jnp.sqrt(ss), 1e-12)
        o_ref[...] = jnp.maximum(pre / n, 0.0)

    return pl.pallas_call(
        k, grid=(ACC0 // 8 // RP,),
        in_specs=[pl.BlockSpec((2, RP, 128), lambda i: (0, i, 0)),
                  pl.BlockSpec((RP, 128), lambda i: (i, 0)),
                  pl.BlockSpec((128, 128), lambda i: (0, 0)),
                  pl.BlockSpec((128, 128), lambda i: (0, 0)),
                  pl.BlockSpec((128, 128), lambda i: (0, 0)),
                  pl.BlockSpec((8, 128), lambda i: (0, 0)),
                  pl.BlockSpec((8, 128), lambda i: (0, 0))],
        out_specs=pl.BlockSpec((RP, 128), lambda i: (i, 0)),
        out_shape=jax.ShapeDtypeStruct((ACC0 // 8, 128), jnp.float32))(
            acc1p, T1p, C6, Sh, S16, MK, B1p)


def _tc_f(acc2p, acc1p, T2p, C6, W2Ap, W2Bp, S48, B2p):
    RP = ACC0 // 8 // 3

    def k(a2_ref, a1_ref, t_ref, c_ref, wa_ref, wb_ref, s_ref, b_ref, o_ref):
        a1 = a1_ref[0] + a1_ref[1]
        inv = 1.0 / jnp.maximum(
            jnp.dot(a1, c_ref[...], preferred_element_type=jnp.float32), 1.0)
        a2 = (a2_ref[0] + a2_ref[1]) * inv
        h = (jnp.dot(t_ref[...], wa_ref[...],
                     preferred_element_type=jnp.float32)
             + jnp.dot(a2, wb_ref[...], preferred_element_type=jnp.float32)
             + b_ref[0:1, :])
        ss = jnp.dot(h * h, s_ref[...], preferred_element_type=jnp.float32)
        n = jnp.maximum(jnp.sqrt(ss), 1e-12)
        h = jnp.maximum(h / n, 0.0)
        col = lax.broadcasted_iota(jnp.int32, h.shape, 1)
        o_ref[...] = jnp.where(col % 48 == 36, 1.0, h)

    return pl.pallas_call(
        k, grid=(ACC0 // 8 // RP,),
        in_specs=[pl.BlockSpec((2, RP, 128), lambda i: (0, i, 0)),
                  pl.BlockSpec((2, RP, 128), lambda i: (0, i, 0)),
                  pl.BlockSpec((RP, 128), lambda i: (i, 0)),
                  pl.BlockSpec((128, 128), lambda i: (0, 0)),
                  pl.BlockSpec((128, 384), lambda i: (0, 0)),
                  pl.BlockSpec((128, 384), lambda i: (0, 0)),
                  pl.BlockSpec((384, 384), lambda i: (0, 0)),
                  pl.BlockSpec((8, 384), lambda i: (0, 0))],
        out_specs=pl.BlockSpec((RP, 384), lambda i: (i, 0)),
        out_shape=jax.ShapeDtypeStruct((ACC0 // 8, 384), jnp.float32))(
            acc2p, acc1p, T2p, C6, W2Ap, W2Bp, S48, B2p)


def _tc_h(acc3, M3, B3, M4):
    R = 2048

    def k(a_ref, m3_ref, b3_ref, m4_ref, o_ref):
        a = a_ref[0] + a_ref[1]
        a = a / jnp.clip(a[:, 36:37], 1.0)
        h3 = jnp.maximum(
            jnp.dot(a, m3_ref[...], preferred_element_type=jnp.float32)
            + b3_ref[0:1, :], 0.0)
        g = jnp.dot(h3, m4_ref[...], preferred_element_type=jnp.float32)
        col = lax.broadcasted_iota(jnp.int32, g.shape, 1)
        o_ref[...] = jnp.where(col == 128, 1.0, g)

    return pl.pallas_call(
        k, grid=(ACC1 // R,),
        in_specs=[pl.BlockSpec((2, R, 48), lambda i: (0, i, 0)),
                  pl.BlockSpec((48, 224), lambda i: (0, 0)),
                  pl.BlockSpec((8, 224), lambda i: (0, 0)),
                  pl.BlockSpec((224, 144), lambda i: (0, 0))],
        out_specs=pl.BlockSpec((R, 144), lambda i: (i, 0)),
        out_shape=jax.ShapeDtypeStruct((ACC1, 144), jnp.float32))(
            acc3, M3, B3, M4)


def _tc_j(acc4, B4):
    def k(a_ref, b_ref, o_ref):
        a = a_ref[0] + a_ref[1]
        cnt = jnp.clip(a[:, 128:129], 1.0)
        o = a[:, 0:128] / cnt + b_ref[0:1, :]
        o_ref[...] = o[0:N2][None]

    return pl.pallas_call(
        k, grid=(1,),
        in_specs=[pl.BlockSpec((2, ACC2, 144), lambda i: (0, 0, 0)),
                  pl.BlockSpec((8, 128), lambda i: (0, 0))],
        out_specs=pl.BlockSpec((1, N2, 128), lambda i: (0, 0, 0)),
        out_shape=jax.ShapeDtypeStruct((1, N2, 128), jnp.float32))(acc4, B4)


def kernel(x, n_id, edge_index0, edge_index1, edge_index2, res_n_id1,
           res_n_id2, W1, b1, W2, b2, W3, b3, W4, b4):
    i32 = jnp.int32
    f32 = jnp.float32
    x2 = x.reshape(x.shape[1], x.shape[2])

    nid_p = jnp.concatenate(
        [n_id.astype(i32),
         jnp.zeros((NID_PAD + 10 * 128 - N0,), i32)]).reshape(-1, 128)

    def pad_e(ei, ep, dummy, extra):
        e = ei.shape[1]
        p = ep + extra * 128
        s = jnp.concatenate([ei[0].astype(i32), jnp.zeros((p - e,), i32)])
        d = jnp.concatenate([ei[1].astype(i32), jnp.full((p - e,), dummy, i32)])
        return s.reshape(-1, 128), d.reshape(-1, 128)

    e0s, e0d = pad_e(edge_index0, E0P, N0, 168)
    e1s, e1d = pad_e(edge_index1, E1P, N1, 48)
    e2s, e2d = pad_e(edge_index2, E2P, N2, 12)
    z0 = jnp.zeros((ACC0, 16), f32)
    z1 = jnp.zeros((ACC1, 48), f32)
    z2 = jnp.zeros((ACC2, 144), f32)

    M1 = jnp.zeros((D, 16), f32).at[:, 0:6].set(W1[D:]).at[:, 8:14].set(W1[:D])
    M3 = jnp.zeros((48, 224), f32).at[0:36, 0:216].set(W3)
    B3 = jnp.zeros((8, 224), f32).at[0, 0:216].set(b3)
    M4 = jnp.zeros((224, 144), f32).at[0:216, 0:128].set(W4)
    B4 = jnp.zeros((8, 128), f32).at[0].set(b4)

    li = jnp.arange(128)
    g16, l16 = li // 16, li % 16
    same16 = g16[:, None] == g16[None, :]
    C6 = jnp.where((l16[:, None] == 6) & same16, 1.0, 0.0)
    Sh = jnp.where(same16 & (l16[:, None] == l16[None, :] + 8)
                   & (l16[None, :] < 6), 1.0, 0.0)
    S16 = jnp.where(same16, 1.0, 0.0)
    MK = jnp.zeros((8, 128), f32).at[0].set(jnp.where(l16 < 6, 1.0, 0.0))
    lj = jnp.arange(384)
    g48, l48 = lj // 48, lj % 48
    P16 = jnp.where(l16[:, None] == jnp.arange(6)[None, :], 1.0, 0.0)
    P48 = jnp.where(l48[:, None] == jnp.arange(36)[None, :], 1.0, 0.0)
    gmask = jnp.where(g16[:, None] == g48[None, :], 1.0, 0.0)
    W2Ap = (P16 @ W2[:6] @ P48.T) * gmask
    W2Bp = (P16 @ W2[6:] @ P48.T) * gmask
    S48 = jnp.where(g48[:, None] == g48[None, :], 1.0, 0.0)
    B1p = jnp.zeros((8, 128), f32).at[0].set(P16 @ b1)
    B2p = jnp.zeros((8, 384), f32).at[0].set(P48 @ b2)

    Y = _tc_proj(x2, M1)
    T1 = _sc_gather(Y, nid_p, 10, 6)
    T1p = T1.reshape(-1, 128)
    acc1 = _sc_segsum(T1, e0s, e0d, z0, ACC0, 16, 168, 72, 6)
    acc1p = acc1.reshape(2, -1, 128)
    T2p = _tc_d(acc1p, T1p, C6, Sh, S16, MK, B1p)
    acc2 = _sc_segsum(T2p.reshape(-1, 16), e0s, e0d, z0, ACC0, 16, 168, 72, 6)
    T3p = _tc_f(acc2.reshape(2, -1, 128), acc1p, T2p, C6, W2Ap, W2Bp, S48, B2p)
    acc3 = _sc_segsum(T3p.reshape(-1, 48), e1s, e1d, z1, ACC1, 48, 48, 16, 4)
    T4 = _tc_h(acc3, M3, B3, M4)
    acc4 = _sc_segsum(T4, e2s, e2d, z2, ACC2, 144, 12, 4, 2)
    return _tc_j(acc4, B4)


# speedup vs baseline: 2.8685x; 1.1159x over previous
import functools

import jax
import jax.numpy as jnp
from jax import lax
from jax.experimental import pallas as pl
from jax.experimental.pallas import tpu as pltpu
from jax.experimental.pallas import tpu_sc as plsc

N0, N1, N2 = 30000, 8000, 2000
D = 128
NID_PAD = 32768
E0P, E1P, E2P = 491520, 131072, 32768
ACC0, ACC1, ACC2 = 30720, 8192, 2048
NW = 32


def _mesh():
    return plsc.VectorSubcoreMesh(core_axis_name="c", subcore_axis_name="s")


def _sc_gather(y, nid2, b0, b1):
    bmax = max(b0, b1)

    @functools.partial(
        pl.kernel, mesh=_mesh(),
        compiler_params=pltpu.CompilerParams(use_tc_tiling_on_sc=False),
        out_type=[jax.ShapeDtypeStruct((NID_PAD, 8), jnp.float32),
                  jax.ShapeDtypeStruct((NID_PAD, 8), jnp.float32)],
        scratch_types=[
            pltpu.VMEM((bmax, 128), jnp.int32),
            pltpu.VMEM((2, 128, 16), jnp.float32),
            pltpu.SemaphoreType.DMA,
            pltpu.SemaphoreType.DMA,
        ])
    def k(y_h, nid_h, tm_h, ts_h, idx_v, rows_v, gsem, ssem):
        cid = lax.axis_index("c")
        sid = lax.axis_index("s")
        off = lax.select(cid == 0, sid * b0, 16 * b0 + sid * b1)
        blocks = lax.select(cid == 0, b0, b1)
        pltpu.sync_copy(nid_h.at[pl.ds(off, bmax)], idx_v)
        pltpu.async_copy(y_h.at[idx_v.at[0]], rows_v.at[0], gsem)

        def store(s, b):
            r = pl.ds((off + b) * 128, 128)
            pltpu.async_copy(rows_v.at[s, :, pl.ds(0, 8)], tm_h.at[r], ssem)
            pltpu.async_copy(rows_v.at[s, :, pl.ds(8, 8)], ts_h.at[r], ssem)

        def store_wait(s, b):
            r = pl.ds((off + b) * 128, 128)
            pltpu.make_async_copy(
                rows_v.at[s, :, pl.ds(0, 8)], tm_h.at[r], ssem).wait()
            pltpu.make_async_copy(
                rows_v.at[s, :, pl.ds(8, 8)], ts_h.at[r], ssem).wait()

        def body(b, c):
            s = lax.rem(b, 2)
            pltpu.make_async_copy(y_h.at[idx_v.at[b]], rows_v.at[s], gsem).wait()

            @pl.when(b > 0)
            def _():
                store_wait(1 - s, b - 1)

            @pl.when(b < blocks - 1)
            def _():
                pltpu.async_copy(y_h.at[idx_v.at[b + 1]], rows_v.at[1 - s], gsem)

            store(s, b)
            return c

        lax.fori_loop(0, blocks, body, 0)
        store_wait(lax.rem(blocks - 1, 2), blocks - 1)

    return k(y, nid2)


def _sc_segsum(msg, src2, dst2, zeros, n_acc, width, b0, b1, nbuf):
    bmax = max(b0, b1)
    rpt = n_acc // 16

    @functools.partial(
        pl.kernel, mesh=_mesh(),
        compiler_params=pltpu.CompilerParams(use_tc_tiling_on_sc=False),
        out_type=jax.ShapeDtypeStruct((2, n_acc, width), jnp.float32),
        scratch_types=[
            pltpu.VMEM((bmax, 128), jnp.int32),
            pltpu.VMEM((bmax, 128), jnp.int32),
            pltpu.VMEM((2, nbuf, 128, width), jnp.float32),
            pltpu.VMEM_SHARED((n_acc, width), jnp.float32),
            pltpu.SemaphoreType.DMA,
            pltpu.SemaphoreType.DMA,
        ])
    def k(msg_h, src_h, dst_h, zero_h, out_h, src_i, dst_i, rows_v, acc_sh,
          gsem, ssem):
        cid = lax.axis_index("c")
        sid = lax.axis_index("s")
        off = lax.select(cid == 0, sid * b0, 16 * b0 + sid * b1)
        groups = lax.select(cid == 0, b0 // nbuf, b1 // nbuf)
        r0 = sid * rpt
        pltpu.sync_copy(src_h.at[pl.ds(off, bmax)], src_i)
        pltpu.sync_copy(dst_h.at[pl.ds(off, bmax)], dst_i)
        pltpu.sync_copy(zero_h.at[pl.ds(r0, rpt)], acc_sh.at[pl.ds(r0, rpt)])
        plsc.subcore_barrier()

        for j in range(nbuf):
            pltpu.async_copy(msg_h.at[src_i.at[j]], rows_v.at[0, j], gsem)

        def giter(g, c):
            s = lax.rem(g, 2)
            base = g * nbuf
            for j in range(nbuf):
                pltpu.make_async_copy(
                    msg_h.at[src_i.at[base + j]], rows_v.at[s, j], gsem).wait()

            @pl.when(g > 0)
            def _():
                for j in range(nbuf):
                    pltpu.make_async_copy(
                        rows_v.at[1 - s, j],
                        acc_sh.at[dst_i.at[base - nbuf + j]], ssem).wait()

            @pl.when(g < groups - 1)
            def _():
                for j in range(nbuf):
                    pltpu.async_copy(
                        msg_h.at[src_i.at[base + nbuf + j]],
                        rows_v.at[1 - s, j], gsem)

            for j in range(nbuf):
                pltpu.async_copy(
                    rows_v.at[s, j], acc_sh.at[dst_i.at[base + j]], ssem,
                    add=True)
            return c

        lax.fori_loop(0, groups, giter, 0)
        sl = lax.rem(groups - 1, 2)
        for j in range(nbuf):
            pltpu.make_async_copy(
                rows_v.at[sl, j],
                acc_sh.at[dst_i.at[(groups - 1) * nbuf + j]], ssem).wait()

        plsc.subcore_barrier()
        pltpu.sync_copy(acc_sh.at[pl.ds(r0, rpt)],
                        out_h.at[cid, pl.ds(r0, rpt)])

    return k(msg, src2, dst2, zeros)


def _tc_proj(x2, M1):
    R = 10000

    def k(x_ref, m_ref, o_ref):
        y = jnp.dot(x_ref[...], m_ref[...], preferred_element_type=jnp.float32)
        col = lax.broadcasted_iota(jnp.int32, y.shape, 1)
        o_ref[...] = jnp.where(col == 6, 1.0, y)

    n = x2.shape[0]
    return pl.pallas_call(
        k, grid=(n // R,),
        in_specs=[pl.BlockSpec((R, D), lambda i: (i, 0)),
                  pl.BlockSpec((D, 16), lambda i: (0, 0))],
        out_specs=pl.BlockSpec((R, 16), lambda i: (i, 0)),
        out_shape=jax.ShapeDtypeStruct((n, 16), jnp.float32))(x2, M1)


def _tc_d(acc1p, T1sp, C6, S8, MK, B1p):
    RP = ACC0 // 16 // 3

    def k(a_ref, t_ref, c_ref, s_ref, mk_ref, b_ref, o_ref):
        a = a_ref[0] + a_ref[1]
        cnt = jnp.maximum(
            jnp.dot(a, c_ref[...], preferred_element_type=jnp.float32), 1.0)
        pre = a / cnt * mk_ref[0:1, :] + t_ref[...] + b_ref[0:1, :]
        ss = jnp.dot(pre * pre, s_ref[...], preferred_element_type=jnp.float32)
        n = jnp.maximum(jnp.sqrt(ss), 1e-12)
        o_ref[...] = jnp.maximum(pre / n, 0.0)

    return pl.pallas_call(
        k, grid=(ACC0 // 16 // RP,),
        in_specs=[pl.BlockSpec((2, RP, 128), lambda i: (0, i, 0)),
                  pl.BlockSpec((RP, 128), lambda i: (i, 0)),
                  pl.BlockSpec((128, 128), lambda i: (0, 0)),
                  pl.BlockSpec((128, 128), lambda i: (0, 0)),
                  pl.BlockSpec((8, 128), lambda i: (0, 0)),
                  pl.BlockSpec((8, 128), lambda i: (0, 0))],
        out_specs=pl.BlockSpec((RP, 128), lambda i: (i, 0)),
        out_shape=jax.ShapeDtypeStruct((ACC0 // 16, 128), jnp.float32))(
            acc1p, T1sp, C6, S8, MK, B1p)


def _tc_f(acc2p, acc1p, T2p, C6, W2Ap, W2Bp, S48, B2p):
    RP = ACC0 // 16 // 3

    def k(a2_ref, a1_ref, t_ref, c_ref, wa_ref, wb_ref, s_ref, b_ref, o_ref):
        a1 = a1_ref[0] + a1_ref[1]
        inv = 1.0 / jnp.maximum(
            jnp.dot(a1, c_ref[...], preferred_element_type=jnp.float32), 1.0)
        a2 = (a2_ref[0] + a2_ref[1]) * inv
        h = (jnp.dot(t_ref[...], wa_ref[...],
                     preferred_element_type=jnp.float32)
             + jnp.dot(a2, wb_ref[...], preferred_element_type=jnp.float32)
             + b_ref[0:1, :])
        ss = jnp.dot(h * h, s_ref[...], preferred_element_type=jnp.float32)
        n = jnp.maximum(jnp.sqrt(ss), 1e-12)
        h = jnp.maximum(h / n, 0.0)
        col = lax.broadcasted_iota(jnp.int32, h.shape, 1)
        o_ref[...] = jnp.where(col % 48 == 36, 1.0, h)

    return pl.pallas_call(
        k, grid=(ACC0 // 16 // RP,),
        in_specs=[pl.BlockSpec((2, RP, 128), lambda i: (0, i, 0)),
                  pl.BlockSpec((2, RP, 128), lambda i: (0, i, 0)),
                  pl.BlockSpec((RP, 128), lambda i: (i, 0)),
                  pl.BlockSpec((128, 128), lambda i: (0, 0)),
                  pl.BlockSpec((128, 768), lambda i: (0, 0)),
                  pl.BlockSpec((128, 768), lambda i: (0, 0)),
                  pl.BlockSpec((768, 768), lambda i: (0, 0)),
                  pl.BlockSpec((8, 768), lambda i: (0, 0))],
        out_specs=pl.BlockSpec((RP, 768), lambda i: (i, 0)),
        out_shape=jax.ShapeDtypeStruct((ACC0 // 16, 768), jnp.float32))(
            acc2p, acc1p, T2p, C6, W2Ap, W2Bp, S48, B2p)


def _tc_h(acc3, M3, B3, M4):
    R = 2048

    def k(a_ref, m3_ref, b3_ref, m4_ref, o_ref):
        a = a_ref[0] + a_ref[1]
        a = a / jnp.clip(a[:, 36:37], 1.0)
        h3 = jnp.maximum(
            jnp.dot(a, m3_ref[...], preferred_element_type=jnp.float32)
            + b3_ref[0:1, :], 0.0)
        g = jnp.dot(h3, m4_ref[...], preferred_element_type=jnp.float32)
        col = lax.broadcasted_iota(jnp.int32, g.shape, 1)
        o_ref[...] = jnp.where(col == 128, 1.0, g)

    return pl.pallas_call(
        k, grid=(ACC1 // R,),
        in_specs=[pl.BlockSpec((2, R, 48), lambda i: (0, i, 0)),
                  pl.BlockSpec((48, 224), lambda i: (0, 0)),
                  pl.BlockSpec((8, 224), lambda i: (0, 0)),
                  pl.BlockSpec((224, 144), lambda i: (0, 0))],
        out_specs=pl.BlockSpec((R, 144), lambda i: (i, 0)),
        out_shape=jax.ShapeDtypeStruct((ACC1, 144), jnp.float32))(
            acc3, M3, B3, M4)


def _tc_j(acc4, B4):
    def k(a_ref, b_ref, o_ref):
        a = a_ref[0] + a_ref[1]
        cnt = jnp.clip(a[:, 128:129], 1.0)
        o = a[:, 0:128] / cnt + b_ref[0:1, :]
        o_ref[...] = o[0:N2][None]

    return pl.pallas_call(
        k, grid=(1,),
        in_specs=[pl.BlockSpec((2, ACC2, 144), lambda i: (0, 0, 0)),
                  pl.BlockSpec((8, 128), lambda i: (0, 0))],
        out_specs=pl.BlockSpec((1, N2, 128), lambda i: (0, 0, 0)),
        out_shape=jax.ShapeDtypeStruct((1, N2, 128), jnp.float32))(acc4, B4)


def kernel(x, n_id, edge_index0, edge_index1, edge_index2, res_n_id1,
           res_n_id2, W1, b1, W2, b2, W3, b3, W4, b4):
    i32 = jnp.int32
    f32 = jnp.float32
    x2 = x.reshape(x.shape[1], x.shape[2])

    nid_p = jnp.concatenate(
        [n_id.astype(i32),
         jnp.zeros((NID_PAD + 10 * 128 - N0,), i32)]).reshape(-1, 128)

    def pad_e(ei, ep, dummy, extra):
        e = ei.shape[1]
        p = ep + extra * 128
        s = jnp.concatenate([ei[0].astype(i32), jnp.zeros((p - e,), i32)])
        d = jnp.concatenate([ei[1].astype(i32), jnp.full((p - e,), dummy, i32)])
        return s.reshape(-1, 128), d.reshape(-1, 128)

    e0s, e0d = pad_e(edge_index0, E0P, N0, 168)
    e1s, e1d = pad_e(edge_index1, E1P, N1, 48)
    e2s, e2d = pad_e(edge_index2, E2P, N2, 12)
    z0 = jnp.zeros((ACC0, 8), f32)
    z1 = jnp.zeros((ACC1, 48), f32)
    z2 = jnp.zeros((ACC2, 144), f32)

    M1 = jnp.zeros((D, 16), f32).at[:, 0:6].set(W1[D:]).at[:, 8:14].set(W1[:D])
    M3 = jnp.zeros((48, 224), f32).at[0:36, 0:216].set(W3)
    B3 = jnp.zeros((8, 224), f32).at[0, 0:216].set(b3)
    M4 = jnp.zeros((224, 144), f32).at[0:216, 0:128].set(W4)
    B4 = jnp.zeros((8, 128), f32).at[0].set(b4)

    li = jnp.arange(128)
    g8, l8 = li // 8, li % 8
    same8 = g8[:, None] == g8[None, :]
    C6 = jnp.where((l8[:, None] == 6) & same8, 1.0, 0.0)
    S8 = jnp.where(same8, 1.0, 0.0)
    MK = jnp.zeros((8, 128), f32).at[0].set(jnp.where(l8 < 6, 1.0, 0.0))
    lj = jnp.arange(768)
    g48, l48 = lj // 48, lj % 48
    P8 = jnp.where(l8[:, None] == jnp.arange(6)[None, :], 1.0, 0.0)
    P48 = jnp.where(l48[:, None] == jnp.arange(36)[None, :], 1.0, 0.0)
    gmask = jnp.where(g8[:, None] == g48[None, :], 1.0, 0.0)
    W2Ap = (P8 @ W2[:6] @ P48.T) * gmask
    W2Bp = (P8 @ W2[6:] @ P48.T) * gmask
    S48 = jnp.where(g48[:, None] == g48[None, :], 1.0, 0.0)
    B1p = jnp.zeros((8, 128), f32).at[0].set(P8 @ b1)
    B2p = jnp.zeros((8, 768), f32).at[0].set(P48 @ b2)

    Y = _tc_proj(x2, M1)
    T1m, T1s = _sc_gather(Y, nid_p, 10, 6)
    T1sp = T1s.reshape(-1, 128)
    acc1 = _sc_segsum(T1m, e0s, e0d, z0, ACC0, 8, 168, 72, 6)
    acc1p = acc1.reshape(2, -1, 128)
    T2p = _tc_d(acc1p, T1sp, C6, S8, MK, B1p)
    acc2 = _sc_segsum(T2p.reshape(-1, 8), e0s, e0d, z0, ACC0, 8, 168, 72, 6)
    T3p = _tc_f(acc2.reshape(2, -1, 128), acc1p, T2p, C6, W2Ap, W2Bp, S48, B2p)
    acc3 = _sc_segsum(T3p.reshape(-1, 48), e1s, e1d, z1, ACC1, 48, 48, 16, 4)
    T4 = _tc_h(acc3, M3, B3, M4)
    acc4 = _sc_segsum(T4, e2s, e2d, z2, ACC2, 144, 12, 4, 2)
    return _tc_j(acc4, B4)


# speedup vs baseline: 2.8931x; 1.0086x over previous
import functools

import jax
import jax.numpy as jnp
from jax import lax
from jax.experimental import pallas as pl
from jax.experimental.pallas import tpu as pltpu
from jax.experimental.pallas import tpu_sc as plsc

N0, N1, N2 = 30000, 8000, 2000
D = 128
NID_PAD = 32768
E0P, E1P, E2P = 491520, 131072, 32768
ACC0, ACC1, ACC2 = 30720, 8192, 2048
NW = 32


def _mesh():
    return plsc.VectorSubcoreMesh(core_axis_name="c", subcore_axis_name="s")


def _sc_gather(y, nid2, b0, b1):
    bmax = max(b0, b1)

    @functools.partial(
        pl.kernel, mesh=_mesh(),
        compiler_params=pltpu.CompilerParams(use_tc_tiling_on_sc=False),
        out_type=[jax.ShapeDtypeStruct((NID_PAD, 8), jnp.float32),
                  jax.ShapeDtypeStruct((NID_PAD, 8), jnp.float32)],
        scratch_types=[
            pltpu.VMEM((bmax, 128), jnp.int32),
            pltpu.VMEM((2, 128, 16), jnp.float32),
            pltpu.SemaphoreType.DMA,
            pltpu.SemaphoreType.DMA,
        ])
    def k(y_h, nid_h, tm_h, ts_h, idx_v, rows_v, gsem, ssem):
        cid = lax.axis_index("c")
        sid = lax.axis_index("s")
        off = lax.select(cid == 0, sid * b0, 16 * b0 + sid * b1)
        blocks = lax.select(cid == 0, b0, b1)
        pltpu.sync_copy(nid_h.at[pl.ds(off, bmax)], idx_v)
        pltpu.async_copy(y_h.at[idx_v.at[0]], rows_v.at[0], gsem)

        def store(s, b):
            r = pl.ds((off + b) * 128, 128)
            pltpu.async_copy(rows_v.at[s, :, pl.ds(0, 8)], tm_h.at[r], ssem)
            pltpu.async_copy(rows_v.at[s, :, pl.ds(8, 8)], ts_h.at[r], ssem)

        def store_wait(s, b):
            r = pl.ds((off + b) * 128, 128)
            pltpu.make_async_copy(
                rows_v.at[s, :, pl.ds(0, 8)], tm_h.at[r], ssem).wait()
            pltpu.make_async_copy(
                rows_v.at[s, :, pl.ds(8, 8)], ts_h.at[r], ssem).wait()

        def body(b, c):
            s = lax.rem(b, 2)
            pltpu.make_async_copy(y_h.at[idx_v.at[b]], rows_v.at[s], gsem).wait()

            @pl.when(b > 0)
            def _():
                store_wait(1 - s, b - 1)

            @pl.when(b < blocks - 1)
            def _():
                pltpu.async_copy(y_h.at[idx_v.at[b + 1]], rows_v.at[1 - s], gsem)

            store(s, b)
            return c

        lax.fori_loop(0, blocks, body, 0)
        store_wait(lax.rem(blocks - 1, 2), blocks - 1)

    return k(y, nid2)


def _sc_segsum(msg, src2, dst2, zeros, n_acc, width, b0, b1, nbuf):
    bmax = max(b0, b1)
    rpt = n_acc // 16

    @functools.partial(
        pl.kernel, mesh=_mesh(),
        compiler_params=pltpu.CompilerParams(use_tc_tiling_on_sc=False),
        out_type=jax.ShapeDtypeStruct((2, n_acc, width), jnp.float32),
        scratch_types=[
            pltpu.VMEM((bmax, 128), jnp.int32),
            pltpu.VMEM((bmax, 128), jnp.int32),
            pltpu.VMEM((2, nbuf, 128, width), jnp.float32),
            pltpu.VMEM_SHARED((n_acc, width), jnp.float32),
            pltpu.SemaphoreType.DMA,
            pltpu.SemaphoreType.DMA,
        ])
    def k(msg_h, src_h, dst_h, zero_h, out_h, src_i, dst_i, rows_v, acc_sh,
          gsem, ssem):
        cid = lax.axis_index("c")
        sid = lax.axis_index("s")
        off = lax.select(cid == 0, sid * b0, 16 * b0 + sid * b1)
        groups = lax.select(cid == 0, b0 // nbuf, b1 // nbuf)
        r0 = sid * rpt
        pltpu.sync_copy(src_h.at[pl.ds(off, bmax)], src_i)
        pltpu.sync_copy(dst_h.at[pl.ds(off, bmax)], dst_i)
        pltpu.sync_copy(zero_h.at[pl.ds(r0, rpt)], acc_sh.at[pl.ds(r0, rpt)])
        plsc.subcore_barrier()

        for j in range(nbuf):
            pltpu.async_copy(msg_h.at[src_i.at[j]], rows_v.at[0, j], gsem)

        def giter(g, c):
            s = lax.rem(g, 2)
            base = g * nbuf
            for j in range(nbuf):
                pltpu.make_async_copy(
                    msg_h.at[src_i.at[base + j]], rows_v.at[s, j], gsem).wait()

            @pl.when(g > 0)
            def _():
                for j in range(nbuf):
                    pltpu.make_async_copy(
                        rows_v.at[1 - s, j],
                        acc_sh.at[dst_i.at[base - nbuf + j]], ssem).wait()

            @pl.when(g < groups - 1)
            def _():
                for j in range(nbuf):
                    pltpu.async_copy(
                        msg_h.at[src_i.at[base + nbuf + j]],
                        rows_v.at[1 - s, j], gsem)

            for j in range(nbuf):
                pltpu.async_copy(
                    rows_v.at[s, j], acc_sh.at[dst_i.at[base + j]], ssem,
                    add=True)
            return c

        lax.fori_loop(0, groups, giter, 0)
        sl = lax.rem(groups - 1, 2)
        for j in range(nbuf):
            pltpu.make_async_copy(
                rows_v.at[sl, j],
                acc_sh.at[dst_i.at[(groups - 1) * nbuf + j]], ssem).wait()

        plsc.subcore_barrier()
        pltpu.sync_copy(acc_sh.at[pl.ds(r0, rpt)],
                        out_h.at[cid, pl.ds(r0, rpt)])

    return k(msg, src2, dst2, zeros)


def _tc_proj(x2, M1):
    R = 10000

    def k(x_ref, m_ref, o_ref):
        y = jnp.dot(x_ref[...], m_ref[...], preferred_element_type=jnp.float32)
        col = lax.broadcasted_iota(jnp.int32, y.shape, 1)
        o_ref[...] = jnp.where(col == 6, 1.0, y)

    n = x2.shape[0]
    return pl.pallas_call(
        k, grid=(n // R,),
        in_specs=[pl.BlockSpec((R, D), lambda i: (i, 0)),
                  pl.BlockSpec((D, 16), lambda i: (0, 0))],
        out_specs=pl.BlockSpec((R, 16), lambda i: (i, 0)),
        out_shape=jax.ShapeDtypeStruct((n, 16), jnp.float32))(x2, M1)


def _tc_d(acc1p, T1sp, C6, S8, MK, B1p):
    RP = ACC0 // 16 // 3

    def k(a_ref, t_ref, c_ref, s_ref, mk_ref, b_ref, o_ref):
        a = a_ref[0] + a_ref[1]
        cnt = jnp.maximum(
            jnp.dot(a, c_ref[...], preferred_element_type=jnp.float32), 1.0)
        pre = a / cnt * mk_ref[0:1, :] + t_ref[...] + b_ref[0:1, :]
        ss = jnp.dot(pre * pre, s_ref[...], preferred_element_type=jnp.float32)
        n = jnp.maximum(jnp.sqrt(ss), 1e-12)
        o_ref[...] = jnp.maximum(pre / n, 0.0)

    return pl.pallas_call(
        k, grid=(ACC0 // 16 // RP,),
        in_specs=[pl.BlockSpec((2, RP, 128), lambda i: (0, i, 0)),
                  pl.BlockSpec((RP, 128), lambda i: (i, 0)),
                  pl.BlockSpec((128, 128), lambda i: (0, 0)),
                  pl.BlockSpec((128, 128), lambda i: (0, 0)),
                  pl.BlockSpec((8, 128), lambda i: (0, 0)),
                  pl.BlockSpec((8, 128), lambda i: (0, 0))],
        out_specs=pl.BlockSpec((RP, 128), lambda i: (i, 0)),
        out_shape=jax.ShapeDtypeStruct((ACC0 // 16, 128), jnp.float32))(
            acc1p, T1sp, C6, S8, MK, B1p)


def _tc_f(acc2p, acc1p, T2p, C6, W2Ap, W2Bp, S48, B2p):
    RP = ACC0 // 16 // 3

    def k(a2_ref, a1_ref, t_ref, c_ref, wa_ref, wb_ref, s_ref, b_ref, o_ref):
        a1 = a1_ref[0] + a1_ref[1]
        inv = 1.0 / jnp.maximum(
            jnp.dot(a1, c_ref[...], preferred_element_type=jnp.float32), 1.0)
        a2 = (a2_ref[0] + a2_ref[1]) * inv
        h = (jnp.dot(t_ref[...], wa_ref[...],
                     preferred_element_type=jnp.float32)
             + jnp.dot(a2, wb_ref[...], preferred_element_type=jnp.float32)
             + b_ref[0:1, :])
        ss = jnp.dot(h * h, s_ref[...], preferred_element_type=jnp.float32)
        n = jnp.maximum(jnp.sqrt(ss), 1e-12)
        h = jnp.maximum(h / n, 0.0)
        col = lax.broadcasted_iota(jnp.int32, h.shape, 1)
        o_ref[...] = jnp.where(col % 48 == 36, 1.0, h)

    return pl.pallas_call(
        k, grid=(ACC0 // 16 // RP,),
        in_specs=[pl.BlockSpec((2, RP, 128), lambda i: (0, i, 0)),
                  pl.BlockSpec((2, RP, 128), lambda i: (0, i, 0)),
                  pl.BlockSpec((RP, 128), lambda i: (i, 0)),
                  pl.BlockSpec((128, 128), lambda i: (0, 0)),
                  pl.BlockSpec((128, 768), lambda i: (0, 0)),
                  pl.BlockSpec((128, 768), lambda i: (0, 0)),
                  pl.BlockSpec((768, 768), lambda i: (0, 0)),
                  pl.BlockSpec((8, 768), lambda i: (0, 0))],
        out_specs=pl.BlockSpec((RP, 768), lambda i: (i, 0)),
        out_shape=jax.ShapeDtypeStruct((ACC0 // 16, 768), jnp.float32))(
            acc2p, acc1p, T2p, C6, W2Ap, W2Bp, S48, B2p)


def _tc_h(acc3, M3, B3, M4):
    R = 2048

    def k(a_ref, m3_ref, b3_ref, m4_ref, o_ref):
        a = a_ref[0] + a_ref[1]
        a = a / jnp.clip(a[:, 36:37], 1.0)
        h3 = jnp.maximum(
            jnp.dot(a, m3_ref[...], preferred_element_type=jnp.float32)
            + b3_ref[0:1, :], 0.0)
        g = jnp.dot(h3, m4_ref[...], preferred_element_type=jnp.float32)
        col = lax.broadcasted_iota(jnp.int32, g.shape, 1)
        o_ref[...] = jnp.where(col == 128, 1.0, g)

    return pl.pallas_call(
        k, grid=(ACC1 // R,),
        in_specs=[pl.BlockSpec((2, R, 48), lambda i: (0, i, 0)),
                  pl.BlockSpec((48, 224), lambda i: (0, 0)),
                  pl.BlockSpec((8, 224), lambda i: (0, 0)),
                  pl.BlockSpec((224, 144), lambda i: (0, 0))],
        out_specs=pl.BlockSpec((R, 144), lambda i: (i, 0)),
        out_shape=jax.ShapeDtypeStruct((ACC1, 144), jnp.float32))(
            acc3, M3, B3, M4)


def _tc_j(acc4, B4):
    def k(a_ref, b_ref, o_ref):
        a = a_ref[0] + a_ref[1]
        cnt = jnp.clip(a[:, 128:129], 1.0)
        o = a[:, 0:128] / cnt + b_ref[0:1, :]
        o_ref[...] = o[0:N2][None]

    return pl.pallas_call(
        k, grid=(1,),
        in_specs=[pl.BlockSpec((2, ACC2, 144), lambda i: (0, 0, 0)),
                  pl.BlockSpec((8, 128), lambda i: (0, 0))],
        out_specs=pl.BlockSpec((1, N2, 128), lambda i: (0, 0, 0)),
        out_shape=jax.ShapeDtypeStruct((1, N2, 128), jnp.float32))(acc4, B4)


def kernel(x, n_id, edge_index0, edge_index1, edge_index2, res_n_id1,
           res_n_id2, W1, b1, W2, b2, W3, b3, W4, b4):
    i32 = jnp.int32
    f32 = jnp.float32
    x2 = x.reshape(x.shape[1], x.shape[2])

    nid_p = jnp.concatenate(
        [n_id.astype(i32),
         jnp.zeros((NID_PAD + 10 * 128 - N0,), i32)]).reshape(-1, 128)

    def pad_e(ei, ep, dummy, extra):
        e = ei.shape[1]
        p = ep + extra * 128
        s = jnp.concatenate([ei[0].astype(i32), jnp.zeros((p - e,), i32)])
        d = jnp.concatenate([ei[1].astype(i32), jnp.full((p - e,), dummy, i32)])
        return s.reshape(-1, 128), d.reshape(-1, 128)

    e0s, e0d = pad_e(edge_index0, E0P, N0, 180)
    e1s, e1d = pad_e(edge_index1, E1P, N1, 52)
    e2s, e2d = pad_e(edge_index2, E2P, N2, 14)
    z0 = jnp.zeros((ACC0, 8), f32)
    z1 = jnp.zeros((ACC1, 48), f32)
    z2 = jnp.zeros((ACC2, 144), f32)

    M1 = jnp.zeros((D, 16), f32).at[:, 0:6].set(W1[D:]).at[:, 8:14].set(W1[:D])
    M3 = jnp.zeros((48, 224), f32).at[0:36, 0:216].set(W3)
    B3 = jnp.zeros((8, 224), f32).at[0, 0:216].set(b3)
    M4 = jnp.zeros((224, 144), f32).at[0:216, 0:128].set(W4)
    B4 = jnp.zeros((8, 128), f32).at[0].set(b4)

    li = jnp.arange(128)
    g8, l8 = li // 8, li % 8
    same8 = g8[:, None] == g8[None, :]
    C6 = jnp.where((l8[:, None] == 6) & same8, 1.0, 0.0)
    S8 = jnp.where(same8, 1.0, 0.0)
    MK = jnp.zeros((8, 128), f32).at[0].set(jnp.where(l8 < 6, 1.0, 0.0))
    lj = jnp.arange(768)
    g48, l48 = lj // 48, lj % 48
    P8 = jnp.where(l8[:, None] == jnp.arange(6)[None, :], 1.0, 0.0)
    P48 = jnp.where(l48[:, None] == jnp.arange(36)[None, :], 1.0, 0.0)
    gmask = jnp.where(g8[:, None] == g48[None, :], 1.0, 0.0)
    W2Ap = (P8 @ W2[:6] @ P48.T) * gmask
    W2Bp = (P8 @ W2[6:] @ P48.T) * gmask
    S48 = jnp.where(g48[:, None] == g48[None, :], 1.0, 0.0)
    B1p = jnp.zeros((8, 128), f32).at[0].set(P8 @ b1)
    B2p = jnp.zeros((8, 768), f32).at[0].set(P48 @ b2)

    Y = _tc_proj(x2, M1)
    T1m, T1s = _sc_gather(Y, nid_p, 10, 6)
    T1sp = T1s.reshape(-1, 128)
    acc1 = _sc_segsum(T1m, e0s, e0d, z0, ACC0, 8, 180, 60, 6)
    acc1p = acc1.reshape(2, -1, 128)
    T2p = _tc_d(acc1p, T1sp, C6, S8, MK, B1p)
    acc2 = _sc_segsum(T2p.reshape(-1, 8), e0s, e0d, z0, ACC0, 8, 180, 60, 6)
    T3p = _tc_f(acc2.reshape(2, -1, 128), acc1p, T2p, C6, W2Ap, W2Bp, S48, B2p)
    acc3 = _sc_segsum(T3p.reshape(-1, 48), e1s, e1d, z1, ACC1, 48, 52, 12, 4)
    T4 = _tc_h(acc3, M3, B3, M4)
    acc4 = _sc_segsum(T4, e2s, e2d, z2, ACC2, 144, 14, 2, 2)
    return _tc_j(acc4, B4)
